# baseline (speedup 1.0000x reference)
import jax
import jax.numpy as jnp
from jax import lax
from jax.experimental import pallas as pl
from jax.experimental.pallas import tpu as pltpu
from jax.experimental.pallas import tpu_sc as plsc

_NC, _NS = 2, 16
_NW = _NC * _NS


_G = 8


_NB = 3


def _sc_gather_body(table_hbm, idx_hbm, out_hbm, idx_v,
                    buf0, buf1, buf2, g0, g1, g2, o0, o1, o2):
    wid = lax.axis_index("s") * _NC + lax.axis_index("c")
    rows_per_w = out_hbm.shape[0] // _NW
    base = wid * rows_per_w
    nchunk = rows_per_w // _G
    pltpu.sync_copy(idx_hbm.at[pl.ds(base, rows_per_w)], idx_v)
    bufs, gsems, osems = (buf0, buf1, buf2), (g0, g1, g2), (o0, o1, o2)

    def gather(k):
        cp = pltpu.make_async_copy(
            table_hbm.at[idx_v.at[pl.ds(k * _G, _G)]],
            bufs[k % _NB], gsems[k % _NB])
        cp.start()
        return cp

    gcps, ocps = [None] * nchunk, [None] * nchunk
    for k in range(min(_NB - 1, nchunk)):
        gcps[k] = gather(k)
    for k in range(nchunk):
        m = k % _NB
        if k + _NB - 1 < nchunk:
            b = (k + _NB - 1) % _NB
            if k - 1 >= 0:
                ocps[k - 1].wait()
            gcps[k + _NB - 1] = gather(k + _NB - 1)
        gcps[m].wait()
        ocps[k] = pltpu.make_async_copy(
            bufs[m], out_hbm.at[pl.ds(base + k * _G, _G)], osems[m])
        ocps[k].start()
    for k in range(max(0, nchunk - _NB), nchunk):
        ocps[k].wait()


def kernel(ts, wMat):
    B, T, C = ts.shape
    I = wMat.shape[0]
    tsT = jnp.swapaxes(ts, 1, 2)
    table = tsT.reshape(B * C, T)
    n = B * I
    j = jnp.arange(n, dtype=jnp.int32)
    idx = C * (j // I) + 4 * (j % I) + 1
    rows_per_w = n // _NW
    mesh = plsc.VectorSubcoreMesh(core_axis_name="c", subcore_axis_name="s")
    out2d = pl.kernel(
        _sc_gather_body,
        out_type=jax.ShapeDtypeStruct((n, T), jnp.float32),
        mesh=mesh,
        scratch_types=(
            [pltpu.VMEM((rows_per_w,), jnp.int32)]
            + [pltpu.VMEM((_G, T), jnp.float32)] * _NB
            + [pltpu.SemaphoreType.DMA] * (2 * _NB)
        ),
        compiler_params=pltpu.CompilerParams(use_tc_tiling_on_sc=True),
    )(table, idx)
    outT = out2d.reshape(B, I, T)
    return jnp.swapaxes(outT, 1, 2)
